# Initial kernel scaffold; baseline (speedup 1.0000x reference)
import jax
import jax.numpy as jnp
from jax.experimental import pallas as pl


def kernel(x, edge_index, orders, W_mp, b_mp, W_out, b_out):
    raise NotImplementedError("write your pallas kernel here")


# speedup vs baseline: 21.0882x; 21.0882x over previous
import functools

import jax
import jax.numpy as jnp
from jax import lax
from jax.experimental import pallas as pl
from jax.experimental.pallas import tpu as pltpu
from jax.experimental.pallas import tpu_sc as plsc

N_NODES = 100000
K = 10
DIM_OUT = 10
GW = 16
N_EDGES = 6400000
EPO = 1600000
N_ORDERS = 4

NC, NS = 2, 16
ROW = 128
NROW = 12544
RPW = NROW // (NC * NS)
NB = 8
N_PAD = 100096
RPT = N_PAD // NS
BN = 6256
DUMP = N_NODES
EPAD = N_EDGES + 64


def _table_from(h, wb):
  g = jnp.dot(h, wb.T, preferred_element_type=jnp.float32)
  ones = jnp.ones((h.shape[0], 1), jnp.float32)
  zer = jnp.zeros((h.shape[0], GW - K - 1), jnp.float32)
  return jnp.concatenate([g, ones, zer], axis=1)


def _build_body(x_ref, w_ref, g_ref):
  g_ref[...] = _table_from(x_ref[...], w_ref[:, K:])


def _new_h(h_ref, a_ref, w_ref, b_ref):
  h = h_ref[...]
  wa = w_ref[:, :K]
  wb = w_ref[:, K:]
  s = a_ref[0] + a_ref[1]
  cnt = s[:, K:K + 1]
  f = jnp.dot(h, (wa - wb).T, preferred_element_type=jnp.float32) + b_ref[...]
  return h + s[:, :K] + cnt * f, wb


def _update_body(h_ref, a_ref, w_ref, b_ref, hn_ref, g_ref):
  hn, wb = _new_h(h_ref, a_ref, w_ref, b_ref)
  hn_ref[...] = hn
  g_ref[...] = _table_from(hn, wb)


def _final_body(h_ref, a_ref, w_ref, b_ref, wo_ref, bo_ref, o_ref):
  hn, _ = _new_h(h_ref, a_ref, w_ref, b_ref)
  o_ref[...] = (jnp.dot(hn, wo_ref[...].T, preferred_element_type=jnp.float32)
                + bo_ref[...])


def _sc_step_body(ord_hbm, ei_dst, ei_src, g_hbm, zeros_hbm, out_hbm,
                  ord_v, dst_v, src_v, rows_v, sem_o, sem_i, sem_g, sem_s,
                  accum):
  c = lax.axis_index("c")
  s = lax.axis_index("s")
  w = c * NS + s
  pltpu.sync_copy(zeros_hbm.at[pl.ds(s * RPT, RPT)],
                  accum.at[pl.ds(s * RPT, RPT)])
  plsc.subcore_barrier()

  row0 = w * RPW

  @pl.loop(0, RPW // NB)
  def _(b):
    r = row0 + b * NB
    pltpu.async_copy(ord_hbm.at[pl.ds(r, NB)], ord_v, sem_o).wait()
    idx = [pltpu.async_copy(ei_dst.at[ord_v.at[j]], dst_v.at[j], sem_i)
           for j in range(NB)]
    idx += [pltpu.async_copy(ei_src.at[ord_v.at[j]], src_v.at[j], sem_i)
            for j in range(NB)]
    for d in idx:
      d.wait()
    gs = [pltpu.async_copy(g_hbm.at[src_v.at[j]], rows_v.at[j], sem_g)
          for j in range(NB)]
    for d in gs:
      d.wait()
    sc = [pltpu.async_copy(rows_v.at[j], accum.at[dst_v.at[j]], sem_s,
                           add=True)
          for j in range(NB)]
    for d in sc:
      d.wait()

  plsc.subcore_barrier()
  pltpu.sync_copy(accum.at[pl.ds(s * RPT, RPT)],
                  out_hbm.at[c].at[pl.ds(s * RPT, RPT)])


def kernel(x, edge_index, orders, W_mp, b_mp, W_out, b_out):
  f32 = jnp.float32
  i32 = jnp.int32
  ei = edge_index.astype(i32)
  ords = orders.astype(i32)
  pad_e = jnp.full((EPAD - N_EDGES,), DUMP, i32)
  ei_dst = jnp.concatenate([ei[0], pad_e])
  ei_src = jnp.concatenate([ei[1], pad_e])
  pad_o = jnp.full((N_ORDERS, NROW * ROW - EPO), N_EDGES, i32)
  ords_p = jnp.concatenate([ords, pad_o], axis=1).reshape(N_ORDERS, NROW, ROW)
  xp = jnp.concatenate([x, jnp.zeros((N_PAD - N_NODES, K), f32)], axis=0)
  zeros_tbl = jnp.zeros((N_PAD, GW), f32)
  b1 = b_mp.reshape(1, K)
  bo1 = b_out.reshape(1, DIM_OUT)

  ngrid = N_PAD // BN
  _hs = pl.BlockSpec((BN, K), lambda i: (i, 0))
  _gs = pl.BlockSpec((BN, GW), lambda i: (i, 0))
  _as = pl.BlockSpec((NC, BN, GW), lambda i: (0, i, 0))
  _ws = pl.BlockSpec((K, 2 * K), lambda i: (0, 0))
  _bs = pl.BlockSpec((1, K), lambda i: (0, 0))
  build = pl.pallas_call(
      _build_body, grid=(ngrid,), in_specs=[_hs, _ws], out_specs=_gs,
      out_shape=jax.ShapeDtypeStruct((N_PAD, GW), f32))
  update = pl.pallas_call(
      _update_body, grid=(ngrid,), in_specs=[_hs, _as, _ws, _bs],
      out_specs=(_hs, _gs),
      out_shape=(jax.ShapeDtypeStruct((N_PAD, K), f32),
                 jax.ShapeDtypeStruct((N_PAD, GW), f32)))
  final = pl.pallas_call(
      _final_body, grid=(ngrid,),
      in_specs=[_hs, _as, _ws, _bs,
                pl.BlockSpec((DIM_OUT, K), lambda i: (0, 0)),
                pl.BlockSpec((1, DIM_OUT), lambda i: (0, 0))],
      out_specs=pl.BlockSpec((BN, DIM_OUT), lambda i: (i, 0)),
      out_shape=jax.ShapeDtypeStruct((N_PAD, DIM_OUT), f32))

  mesh = plsc.VectorSubcoreMesh(core_axis_name="c", subcore_axis_name="s")
  sc_step = functools.partial(
      pl.kernel,
      out_type=jax.ShapeDtypeStruct((NC, N_PAD, GW), f32),
      mesh=mesh,
      compiler_params=pltpu.CompilerParams(use_tc_tiling_on_sc=False),
      scratch_types=[
          pltpu.VMEM((NB, ROW), i32),
          pltpu.VMEM((NB, ROW), i32),
          pltpu.VMEM((NB, ROW), i32),
          pltpu.VMEM((NB, ROW, GW), f32),
          pltpu.SemaphoreType.DMA,
          pltpu.SemaphoreType.DMA,
          pltpu.SemaphoreType.DMA,
          pltpu.SemaphoreType.DMA,
          pltpu.VMEM_SHARED((N_PAD, GW), f32),
      ],
  )(_sc_step_body)

  h = xp
  g = build(xp, W_mp)
  for i in range(N_ORDERS - 1):
    a = sc_step(ords_p[i], ei_dst, ei_src, g, zeros_tbl)
    h, g = update(h, a, W_mp, b1)
  a = sc_step(ords_p[N_ORDERS - 1], ei_dst, ei_src, g, zeros_tbl)
  out = final(h, a, W_mp, b1, W_out, bo1)
  return out[:N_NODES]


# speedup vs baseline: 22.0202x; 1.0442x over previous
import functools

import jax
import jax.numpy as jnp
from jax import lax
from jax.experimental import pallas as pl
from jax.experimental.pallas import tpu as pltpu
from jax.experimental.pallas import tpu_sc as plsc

N_NODES = 100000
K = 10
DIM_OUT = 10
GW = 16
N_EDGES = 6400000
EPO = 1600000
N_ORDERS = 4

NC, NS = 2, 16
ROW = 512
NROW = 3136
RPW = NROW // (NC * NS)
HB = RPW // 2
N_PAD = 100096
RPT = N_PAD // NS
BN = 6256
DUMP = N_NODES
EPAD = N_EDGES + 64

_PREC = lax.Precision.HIGHEST


def _table_from(h, wb):
  g = jnp.dot(h, wb.T, preferred_element_type=jnp.float32, precision=_PREC)
  ones = jnp.ones((h.shape[0], 1), jnp.float32)
  zer = jnp.zeros((h.shape[0], GW - K - 1), jnp.float32)
  return jnp.concatenate([g, ones, zer], axis=1)


def _build_body(x_ref, w_ref, g_ref):
  g_ref[...] = _table_from(x_ref[...], w_ref[:, K:])


def _new_h(h_ref, a_ref, w_ref, b_ref):
  h = h_ref[...]
  wa = w_ref[:, :K]
  wb = w_ref[:, K:]
  s = a_ref[0] + a_ref[1]
  cnt = s[:, K:K + 1]
  f = jnp.dot(h, (wa - wb).T, preferred_element_type=jnp.float32,
              precision=_PREC) + b_ref[...]
  return h + s[:, :K] + cnt * f, wb


def _update_body(h_ref, a_ref, w_ref, b_ref, hn_ref, g_ref):
  hn, wb = _new_h(h_ref, a_ref, w_ref, b_ref)
  hn_ref[...] = hn
  g_ref[...] = _table_from(hn, wb)


def _final_body(h_ref, a_ref, w_ref, b_ref, wo_ref, bo_ref, o_ref):
  hn, _ = _new_h(h_ref, a_ref, w_ref, b_ref)
  o_ref[...] = (jnp.dot(hn, wo_ref[...].T, preferred_element_type=jnp.float32,
                        precision=_PREC) + bo_ref[...])


def _sc_step_body(ord_hbm, ei_dst, ei_src, g_hbm, zeros_hbm, out_hbm,
                  ord0, ord1, dst0, dst1, src0, src1, rows0, rows1,
                  so0, so1, si0, si1, sg0, sg1, ss0, ss1, accum):
  c = lax.axis_index("c")
  s = lax.axis_index("s")
  w = c * NS + s

  pltpu.sync_copy(zeros_hbm.at[pl.ds(s * RPT, RPT)],
                  accum.at[pl.ds(s * RPT, RPT)])
  plsc.subcore_barrier()

  row0 = w * RPW

  def o_start(b, ordv, sem):
    pltpu.async_copy(ord_hbm.at[row0 + b], ordv, sem)

  def o_wait(ordv, sem):
    pltpu.make_async_copy(ord_hbm.at[0], ordv, sem).wait()

  def idx_start(ordv, dstv, srcv, sem):
    pltpu.async_copy(ei_dst.at[ordv], dstv, sem)
    pltpu.async_copy(ei_src.at[ordv], srcv, sem)

  def idx_wait(dstv, srcv, sem):
    pltpu.make_async_copy(ei_dst.at[dst0], dstv, sem).wait()
    pltpu.make_async_copy(ei_src.at[dst0], srcv, sem).wait()

  def g_start(srcv, rowsv, sem):
    pltpu.async_copy(g_hbm.at[srcv], rowsv, sem)

  def g_wait(srcv, rowsv, sem):
    pltpu.make_async_copy(g_hbm.at[srcv], rowsv, sem).wait()

  def s_start(rowsv, dstv, sem):
    pltpu.async_copy(rowsv, accum.at[dstv], sem, add=True)

  def s_wait(rowsv, dstv, sem):
    pltpu.make_async_copy(rowsv, accum.at[dstv], sem).wait()

  o_start(0, ord0, so0)
  o_start(1, ord1, so1)
  o_wait(ord0, so0)
  idx_start(ord0, dst0, src0, si0)

  @pl.loop(0, HB)
  def _(i):
    a = 2 * i
    b = a + 1
    idx_wait(dst0, src0, si0)
    g_start(src0, rows0, sg0)

    @pl.when(a + 2 < RPW)
    def _():
      o_start(a + 2, ord0, so0)

    @pl.when(i > 0)
    def _():
      s_wait(rows1, dst1, ss1)

    o_wait(ord1, so1)
    idx_start(ord1, dst1, src1, si1)
    g_wait(src0, rows0, sg0)
    s_start(rows0, dst0, ss0)
    idx_wait(dst1, src1, si1)
    g_start(src1, rows1, sg1)

    @pl.when(b + 2 < RPW)
    def _():
      o_start(b + 2, ord1, so1)

    s_wait(rows0, dst0, ss0)

    @pl.when(a + 2 < RPW)
    def _():
      o_wait(ord0, so0)
      idx_start(ord0, dst0, src0, si0)

    g_wait(src1, rows1, sg1)
    s_start(rows1, dst1, ss1)

  s_wait(rows1, dst1, ss1)

  plsc.subcore_barrier()
  pltpu.sync_copy(accum.at[pl.ds(s * RPT, RPT)],
                  out_hbm.at[c].at[pl.ds(s * RPT, RPT)])


def kernel(x, edge_index, orders, W_mp, b_mp, W_out, b_out):
  f32 = jnp.float32
  i32 = jnp.int32
  ei = edge_index.astype(i32)
  ords = orders.astype(i32)
  pad_e = jnp.full((EPAD - N_EDGES,), DUMP, i32)
  ei_dst = jnp.concatenate([ei[0], pad_e])
  ei_src = jnp.concatenate([ei[1], pad_e])
  pad_o = jnp.full((N_ORDERS, NROW * ROW - EPO), N_EDGES, i32)
  ords_p = jnp.concatenate([ords, pad_o], axis=1).reshape(N_ORDERS, NROW, ROW)
  xp = jnp.concatenate([x, jnp.zeros((N_PAD - N_NODES, K), f32)], axis=0)
  zeros_tbl = jnp.zeros((N_PAD, GW), f32)
  b1 = b_mp.reshape(1, K)
  bo1 = b_out.reshape(1, DIM_OUT)

  ngrid = N_PAD // BN
  _hs = pl.BlockSpec((BN, K), lambda i: (i, 0))
  _gs = pl.BlockSpec((BN, GW), lambda i: (i, 0))
  _as = pl.BlockSpec((NC, BN, GW), lambda i: (0, i, 0))
  _ws = pl.BlockSpec((K, 2 * K), lambda i: (0, 0))
  _bs = pl.BlockSpec((1, K), lambda i: (0, 0))
  build = pl.pallas_call(
      _build_body, grid=(ngrid,), in_specs=[_hs, _ws], out_specs=_gs,
      out_shape=jax.ShapeDtypeStruct((N_PAD, GW), f32))
  update = pl.pallas_call(
      _update_body, grid=(ngrid,), in_specs=[_hs, _as, _ws, _bs],
      out_specs=(_hs, _gs),
      out_shape=(jax.ShapeDtypeStruct((N_PAD, K), f32),
                 jax.ShapeDtypeStruct((N_PAD, GW), f32)))
  final = pl.pallas_call(
      _final_body, grid=(ngrid,),
      in_specs=[_hs, _as, _ws, _bs,
                pl.BlockSpec((DIM_OUT, K), lambda i: (0, 0)),
                pl.BlockSpec((1, DIM_OUT), lambda i: (0, 0))],
      out_specs=pl.BlockSpec((BN, DIM_OUT), lambda i: (i, 0)),
      out_shape=jax.ShapeDtypeStruct((N_PAD, DIM_OUT), f32))

  mesh = plsc.VectorSubcoreMesh(core_axis_name="c", subcore_axis_name="s")
  sc_step = functools.partial(
      pl.kernel,
      out_type=jax.ShapeDtypeStruct((NC, N_PAD, GW), f32),
      mesh=mesh,
      compiler_params=pltpu.CompilerParams(use_tc_tiling_on_sc=False),
      scratch_types=(
          [pltpu.VMEM((ROW,), i32) for _ in range(6)]
          + [pltpu.VMEM((ROW, GW), f32) for _ in range(2)]
          + [pltpu.SemaphoreType.DMA for _ in range(8)]
          + [pltpu.VMEM_SHARED((N_PAD, GW), f32)]
      ),
  )(_sc_step_body)

  h = xp
  g = build(xp, W_mp)
  for i in range(N_ORDERS - 1):
    a = sc_step(ords_p[i], ei_dst, ei_src, g, zeros_tbl)
    h, g = update(h, a, W_mp, b1)
  a = sc_step(ords_p[N_ORDERS - 1], ei_dst, ei_src, g, zeros_tbl)
  out = final(h, a, W_mp, b1, W_out, bo1)
  return out[:N_NODES]
